# Initial kernel scaffold; baseline (speedup 1.0000x reference)
import functools

import jax
import jax.numpy as jnp
from jax import lax
from jax.experimental import pallas as pl
from jax.experimental.pallas import tpu as pltpu
from jax.experimental.pallas import tpu_sc as plsc

N_NODES = 10000
N_EDGES = 320000
D = 128

NC = 2
NS = 16
NW = NC * NS
EPW = N_EDGES // NW
C = 80
G = EPW // C
RPT = N_NODES // NS
ZR = 125


def _make_sc_seg_sum(compute_deg: bool):
    out_type = [jax.ShapeDtypeStruct((NC, N_NODES, D), jnp.float32)]
    if compute_deg:
        out_type.append(jax.ShapeDtypeStruct((NW, N_NODES), jnp.float32))

    scratch = [
        pltpu.VMEM((G, C), jnp.int32),
        pltpu.VMEM((G, C), jnp.int32),
        pltpu.VMEM((C, D), jnp.float32),
        pltpu.VMEM((ZR, D), jnp.float32),
        pltpu.VMEM_SHARED((N_NODES, D), jnp.float32),
        pltpu.SemaphoreType.DMA,
    ]
    if compute_deg:
        scratch.insert(4, pltpu.VMEM((N_NODES,), jnp.float32))

    mesh = plsc.VectorSubcoreMesh(core_axis_name="c", subcore_axis_name="s")

    def body(h_hbm, src_hbm, dst_hbm, *rest):
        if compute_deg:
            (aggp_hbm, degp_hbm, idx_s, idx_d, rows, zrows, deg, acc,
             sem) = rest
        else:
            aggp_hbm, idx_s, idx_d, rows, zrows, acc, sem = rest
        cid = lax.axis_index("c")
        sid = lax.axis_index("s")
        wid = cid * NS + sid

        zero16 = jnp.zeros((16,), jnp.float32)

        def zrow(r, carry):
            for j in range(D // 16):
                zrows[r, pl.ds(j * 16, 16)] = zero16
            return carry

        lax.fori_loop(0, ZR, zrow, 0)
        for z in range(RPT // ZR):
            pltpu.sync_copy(zrows, acc.at[pl.ds(sid * RPT + z * ZR, ZR)])
        if compute_deg:
            def zdeg(r, carry):
                deg[pl.ds(r * 16, 16)] = zero16
                return carry

            lax.fori_loop(0, N_NODES // 16, zdeg, 0)

        pltpu.sync_copy(src_hbm.at[wid], idx_s)
        pltpu.sync_copy(dst_hbm.at[wid], idx_d)
        plsc.subcore_barrier()

        ones16 = jnp.ones((16,), jnp.float32)

        def chunk(g, carry):
            pltpu.async_copy(h_hbm.at[idx_s.at[g]], rows, sem).wait()
            pltpu.sync_copy(rows, acc.at[idx_d.at[g]], add=True)
            if compute_deg:
                for j in range(C // 16):
                    i16 = idx_d[g, pl.ds(j * 16, 16)]
                    plsc.addupdate_scatter(deg, [i16], ones16)
            return carry

        lax.fori_loop(0, G, chunk, 0)
        plsc.subcore_barrier()

        pltpu.sync_copy(acc.at[pl.ds(sid * RPT, RPT)],
                        aggp_hbm.at[cid, pl.ds(sid * RPT, RPT)])
        if compute_deg:
            pltpu.sync_copy(deg, degp_hbm.at[wid])

    return pl.kernel(body, out_type=out_type, mesh=mesh,
                     scratch_types=scratch)


_sc_seg_sum_deg = _make_sc_seg_sum(True)
_sc_seg_sum = _make_sc_seg_sum(False)

BN = 1000


def _tc_layer_body(h_ref, aggp_ref, degp_ref, ws_ref, wn_ref, b_ref,
                   out_ref):
    agg = aggp_ref[0] + aggp_ref[1]
    deg = jnp.sum(degp_ref[...], axis=0)
    inv = 1.0 / jnp.maximum(deg, 1.0)
    hn = agg * inv[:, None]
    out = (jnp.dot(h_ref[...], ws_ref[...],
                   preferred_element_type=jnp.float32)
           + jnp.dot(hn, wn_ref[...], preferred_element_type=jnp.float32)
           + b_ref[...])
    out_ref[...] = jnp.maximum(out, 0.0)


def _tc_layer(h, aggp, degp, W_self, W_neigh, b):
    return pl.pallas_call(
        _tc_layer_body,
        grid=(N_NODES // BN,),
        in_specs=[
            pl.BlockSpec((BN, D), lambda i: (i, 0)),
            pl.BlockSpec((NC, BN, D), lambda i: (0, i, 0)),
            pl.BlockSpec((NW, BN), lambda i: (0, i)),
            pl.BlockSpec((D, D), lambda i: (0, 0)),
            pl.BlockSpec((D, D), lambda i: (0, 0)),
            pl.BlockSpec((1, D), lambda i: (0, 0)),
        ],
        out_specs=pl.BlockSpec((BN, D), lambda i: (i, 0)),
        out_shape=jax.ShapeDtypeStruct((N_NODES, D), jnp.float32),
    )(h, aggp, degp, W_self, W_neigh, b.reshape(1, D))


def kernel(in_feat, edge_index, W_self1, W_neigh1, b1, W_self2, W_neigh2,
           b2):
    edge_index = edge_index.astype(jnp.int32)
    src3 = edge_index[0].reshape(NW, G, C)
    dst3 = edge_index[1].reshape(NW, G, C)
    h = in_feat.astype(jnp.float32)

    aggp1, degp = _sc_seg_sum_deg(h, src3, dst3)
    h1 = _tc_layer(h, aggp1, degp, W_self1, W_neigh1, b1)
    (aggp2,) = _sc_seg_sum(h1, src3, dst3)
    out = _tc_layer(h1, aggp2, degp, W_self2, W_neigh2, b2)
    return out


# speedup vs baseline: 7.3385x; 7.3385x over previous
import jax
import jax.numpy as jnp
from jax import lax
from jax.experimental import pallas as pl
from jax.experimental.pallas import tpu as pltpu
from jax.experimental.pallas import tpu_sc as plsc

N_NODES = 10000
N_EDGES = 320000
D = 128
DA = 136

NC = 2
NS = 16
NW = NC * NS
EPW = N_EDGES // NW
C = 80
G = EPW // C
ZR = 80


def _make_sc_seg_sum(W: int):
    out_type = jax.ShapeDtypeStruct((NC, N_NODES, W), jnp.float32)
    scratch = [
        pltpu.VMEM((G, C), jnp.int32),
        pltpu.VMEM((G, C), jnp.int32),
        pltpu.VMEM((C, W), jnp.float32),
        pltpu.VMEM((ZR, W), jnp.float32),
        pltpu.VMEM_SHARED((N_NODES, W), jnp.float32),
        pltpu.SemaphoreType.DMA,
    ]
    mesh = plsc.VectorSubcoreMesh(core_axis_name="c", subcore_axis_name="s")

    def body(h_hbm, src_hbm, dst_hbm, aggp_hbm, idx_s, idx_d, rows, zrows,
             acc, sem):
        cid = lax.axis_index("c")
        sid = lax.axis_index("s")
        wid = cid * NS + sid

        zero16 = jnp.zeros((16,), jnp.float32)

        def zrow(r, carry):
            for j in range(W // 16):
                zrows[r, pl.ds(j * 16, 16)] = zero16
            return carry

        lax.fori_loop(0, ZR, zrow, 0)

        @pl.when(sid < NS - 1)
        def _zero_big():
            def zb(z, carry):
                pltpu.sync_copy(zrows,
                                acc.at[pl.ds(sid * 640 + z * ZR, ZR)])
                return carry

            lax.fori_loop(0, 640 // ZR, zb, 0)

        @pl.when(sid == NS - 1)
        def _zero_tail():
            def zb(z, carry):
                pltpu.sync_copy(
                    zrows, acc.at[pl.ds((NS - 1) * 640 + z * ZR, ZR)])
                return carry

            lax.fori_loop(0, 400 // ZR, zb, 0)

        pltpu.sync_copy(src_hbm.at[wid], idx_s)
        pltpu.sync_copy(dst_hbm.at[wid], idx_d)
        plsc.subcore_barrier()

        def chunk(g, carry):
            pltpu.async_copy(h_hbm.at[idx_s.at[g]], rows, sem).wait()
            pltpu.sync_copy(rows, acc.at[idx_d.at[g]], add=True)
            return carry

        lax.fori_loop(0, G, chunk, 0)
        plsc.subcore_barrier()

        @pl.when(sid < NS - 1)
        def _write_big():
            pltpu.sync_copy(acc.at[pl.ds(sid * 640, 640)],
                            aggp_hbm.at[cid, pl.ds(sid * 640, 640)])

        @pl.when(sid == NS - 1)
        def _write_tail():
            pltpu.sync_copy(acc.at[pl.ds((NS - 1) * 640, 400)],
                            aggp_hbm.at[cid, pl.ds((NS - 1) * 640, 400)])

    return pl.kernel(
        body, out_type=out_type, mesh=mesh, scratch_types=scratch,
        compiler_params=pltpu.CompilerParams(use_tc_tiling_on_sc=False))


_sc_seg_sum_aug = _make_sc_seg_sum(DA)
_sc_seg_sum = _make_sc_seg_sum(D)

BN = 1000


def _tc1_body(h_ref, aggp_ref, ws_ref, wn_ref, b_ref, out_ref, invd_ref):
    agg = aggp_ref[0] + aggp_ref[1]
    feat = agg[:, :D]
    deg = agg[:, D]
    inv = 1.0 / jnp.maximum(deg, 1.0)
    hn = feat * inv[:, None]
    out = (jnp.dot(h_ref[...], ws_ref[...],
                   preferred_element_type=jnp.float32)
           + jnp.dot(hn, wn_ref[...], preferred_element_type=jnp.float32)
           + b_ref[...])
    out_ref[...] = jnp.maximum(out, 0.0)
    invd_ref[...] = jnp.broadcast_to(inv[:, None], (BN, 8))


def _tc1(h, aggp, W_self, W_neigh, b):
    return pl.pallas_call(
        _tc1_body,
        grid=(N_NODES // BN,),
        in_specs=[
            pl.BlockSpec((BN, D), lambda i: (i, 0)),
            pl.BlockSpec((NC, BN, DA), lambda i: (0, i, 0)),
            pl.BlockSpec((D, D), lambda i: (0, 0)),
            pl.BlockSpec((D, D), lambda i: (0, 0)),
            pl.BlockSpec((1, D), lambda i: (0, 0)),
        ],
        out_specs=[
            pl.BlockSpec((BN, D), lambda i: (i, 0)),
            pl.BlockSpec((BN, 8), lambda i: (i, 0)),
        ],
        out_shape=[
            jax.ShapeDtypeStruct((N_NODES, D), jnp.float32),
            jax.ShapeDtypeStruct((N_NODES, 8), jnp.float32),
        ],
    )(h, aggp, W_self, W_neigh, b.reshape(1, D))


def _tc2_body(h_ref, aggp_ref, invd_ref, ws_ref, wn_ref, b_ref, out_ref):
    agg = aggp_ref[0] + aggp_ref[1]
    inv = invd_ref[:, 0]
    hn = agg * inv[:, None]
    out = (jnp.dot(h_ref[...], ws_ref[...],
                   preferred_element_type=jnp.float32)
           + jnp.dot(hn, wn_ref[...], preferred_element_type=jnp.float32)
           + b_ref[...])
    out_ref[...] = jnp.maximum(out, 0.0)


def _tc2(h, aggp, invd, W_self, W_neigh, b):
    return pl.pallas_call(
        _tc2_body,
        grid=(N_NODES // BN,),
        in_specs=[
            pl.BlockSpec((BN, D), lambda i: (i, 0)),
            pl.BlockSpec((NC, BN, D), lambda i: (0, i, 0)),
            pl.BlockSpec((BN, 8), lambda i: (i, 0)),
            pl.BlockSpec((D, D), lambda i: (0, 0)),
            pl.BlockSpec((D, D), lambda i: (0, 0)),
            pl.BlockSpec((1, D), lambda i: (0, 0)),
        ],
        out_specs=pl.BlockSpec((BN, D), lambda i: (i, 0)),
        out_shape=jax.ShapeDtypeStruct((N_NODES, D), jnp.float32),
    )(h, aggp, invd, W_self, W_neigh, b.reshape(1, D))


def kernel(in_feat, edge_index, W_self1, W_neigh1, b1, W_self2, W_neigh2,
           b2):
    edge_index = edge_index.astype(jnp.int32)
    src3 = edge_index[0].reshape(NW, G, C)
    dst3 = edge_index[1].reshape(NW, G, C)
    h = in_feat.astype(jnp.float32)
    haug = jnp.concatenate(
        [h, jnp.ones((N_NODES, 1), jnp.float32),
         jnp.zeros((N_NODES, DA - D - 1), jnp.float32)], axis=1)

    aggp1 = _sc_seg_sum_aug(haug, src3, dst3)
    h1, invd = _tc1(h, aggp1, W_self1, W_neigh1, b1)
    aggp2 = _sc_seg_sum(h1, src3, dst3)
    out = _tc2(h1, aggp2, invd, W_self2, W_neigh2, b2)
    return out


# speedup vs baseline: 12.6940x; 1.7298x over previous
import jax
import jax.numpy as jnp
from jax import lax
from jax.experimental import pallas as pl
from jax.experimental.pallas import tpu as pltpu
from jax.experimental.pallas import tpu_sc as plsc

N_NODES = 10000
N_EDGES = 320000
D = 128
DA = 136

NC = 2
NS = 16
NW = NC * NS
EPW = N_EDGES // NW
C = 40
G = EPW // C
NBUF = 4


def _make_sc_seg_sum(W: int):
    out_type = jax.ShapeDtypeStruct((NC, N_NODES, W), jnp.float32)
    scratch = [
        pltpu.VMEM((G, C), jnp.int32),
        pltpu.VMEM((G, C), jnp.int32),
        [pltpu.VMEM((C, W), jnp.float32) for _ in range(NBUF)],
        pltpu.VMEM_SHARED((N_NODES, W), jnp.float32),
        [pltpu.SemaphoreType.DMA for _ in range(NBUF)],
    ]
    mesh = plsc.VectorSubcoreMesh(core_axis_name="c", subcore_axis_name="s")

    zoff = sorted({j * 16 for j in range(W // 16)} | {W - 16})

    def body(h_hbm, src_hbm, dst_hbm, aggp_hbm, idx_s, idx_d, rows, acc,
             sems):
        cid = lax.axis_index("c")
        sid = lax.axis_index("s")
        wid = cid * NS + sid

        zero16 = jnp.zeros((16,), jnp.float32)

        def zrow(r, carry):
            for j in zoff:
                rows[0][r, pl.ds(j, 16)] = zero16
            return carry

        lax.fori_loop(0, C, zrow, 0)

        def zb(z, carry):
            pltpu.sync_copy(rows[0], acc.at[pl.ds(sid * 640 + z * C, C)])
            return carry

        @pl.when(sid < NS - 1)
        def _zero_big():
            lax.fori_loop(0, 640 // C, zb, 0)

        @pl.when(sid == NS - 1)
        def _zero_tail():
            lax.fori_loop(0, 400 // C, zb, 0)

        pltpu.sync_copy(src_hbm.at[wid], idx_s)
        pltpu.sync_copy(dst_hbm.at[wid], idx_d)
        plsc.subcore_barrier()

        for b in range(NBUF):
            pltpu.async_copy(h_hbm.at[idx_s.at[b]], rows[b], sems[b])

        def step(c, b):
            pltpu.make_async_copy(h_hbm.at[idx_s.at[c]], rows[b],
                                  sems[b]).wait()
            pltpu.sync_copy(rows[b], acc.at[idx_d.at[c]], add=True)

            @pl.when(c + NBUF < G)
            def _prefetch():
                pltpu.async_copy(h_hbm.at[idx_s.at[c + NBUF]], rows[b],
                                 sems[b])

        def rnd(r, carry):
            for b in range(NBUF):
                step(r * NBUF + b, b)
            return carry

        lax.fori_loop(0, G // NBUF, rnd, 0)
        for t in range(G - NBUF * (G // NBUF)):
            step(NBUF * (G // NBUF) + t, t)
        plsc.subcore_barrier()

        @pl.when(sid < NS - 1)
        def _write_big():
            pltpu.sync_copy(acc.at[pl.ds(sid * 640, 640)],
                            aggp_hbm.at[cid, pl.ds(sid * 640, 640)])

        @pl.when(sid == NS - 1)
        def _write_tail():
            pltpu.sync_copy(acc.at[pl.ds((NS - 1) * 640, 400)],
                            aggp_hbm.at[cid, pl.ds((NS - 1) * 640, 400)])

    return pl.kernel(
        body, out_type=out_type, mesh=mesh, scratch_types=scratch,
        compiler_params=pltpu.CompilerParams(use_tc_tiling_on_sc=False))


_sc_seg_sum_aug = _make_sc_seg_sum(DA)
_sc_seg_sum = _make_sc_seg_sum(D)

BN = 1000


def _tc1_body(h_ref, aggp_ref, ws_ref, wn_ref, b_ref, out_ref, invd_ref):
    agg = aggp_ref[0] + aggp_ref[1]
    feat = agg[:, :D]
    deg = agg[:, D]
    inv = 1.0 / jnp.maximum(deg, 1.0)
    hn = feat * inv[:, None]
    out = (jnp.dot(h_ref[...], ws_ref[...],
                   preferred_element_type=jnp.float32)
           + jnp.dot(hn, wn_ref[...], preferred_element_type=jnp.float32)
           + b_ref[...])
    out_ref[...] = jnp.maximum(out, 0.0)
    invd_ref[...] = jnp.broadcast_to(inv[:, None], (BN, 8))


def _tc1(h, aggp, W_self, W_neigh, b):
    return pl.pallas_call(
        _tc1_body,
        grid=(N_NODES // BN,),
        in_specs=[
            pl.BlockSpec((BN, D), lambda i: (i, 0)),
            pl.BlockSpec((NC, BN, DA), lambda i: (0, i, 0)),
            pl.BlockSpec((D, D), lambda i: (0, 0)),
            pl.BlockSpec((D, D), lambda i: (0, 0)),
            pl.BlockSpec((1, D), lambda i: (0, 0)),
        ],
        out_specs=[
            pl.BlockSpec((BN, D), lambda i: (i, 0)),
            pl.BlockSpec((BN, 8), lambda i: (i, 0)),
        ],
        out_shape=[
            jax.ShapeDtypeStruct((N_NODES, D), jnp.float32),
            jax.ShapeDtypeStruct((N_NODES, 8), jnp.float32),
        ],
    )(h, aggp, W_self, W_neigh, b.reshape(1, D))


def _tc2_body(h_ref, aggp_ref, invd_ref, ws_ref, wn_ref, b_ref, out_ref):
    agg = aggp_ref[0] + aggp_ref[1]
    inv = invd_ref[:, 0]
    hn = agg * inv[:, None]
    out = (jnp.dot(h_ref[...], ws_ref[...],
                   preferred_element_type=jnp.float32)
           + jnp.dot(hn, wn_ref[...], preferred_element_type=jnp.float32)
           + b_ref[...])
    out_ref[...] = jnp.maximum(out, 0.0)


def _tc2(h, aggp, invd, W_self, W_neigh, b):
    return pl.pallas_call(
        _tc2_body,
        grid=(N_NODES // BN,),
        in_specs=[
            pl.BlockSpec((BN, D), lambda i: (i, 0)),
            pl.BlockSpec((NC, BN, D), lambda i: (0, i, 0)),
            pl.BlockSpec((BN, 8), lambda i: (i, 0)),
            pl.BlockSpec((D, D), lambda i: (0, 0)),
            pl.BlockSpec((D, D), lambda i: (0, 0)),
            pl.BlockSpec((1, D), lambda i: (0, 0)),
        ],
        out_specs=pl.BlockSpec((BN, D), lambda i: (i, 0)),
        out_shape=jax.ShapeDtypeStruct((N_NODES, D), jnp.float32),
    )(h, aggp, invd, W_self, W_neigh, b.reshape(1, D))


def kernel(in_feat, edge_index, W_self1, W_neigh1, b1, W_self2, W_neigh2,
           b2):
    edge_index = edge_index.astype(jnp.int32)
    src3 = edge_index[0].reshape(NW, G, C)
    dst3 = edge_index[1].reshape(NW, G, C)
    h = in_feat.astype(jnp.float32)
    haug = jnp.concatenate(
        [h, jnp.ones((N_NODES, 1), jnp.float32),
         jnp.zeros((N_NODES, DA - D - 1), jnp.float32)], axis=1)

    aggp1 = _sc_seg_sum_aug(haug, src3, dst3)
    h1, invd = _tc1(h, aggp1, W_self1, W_neigh1, b1)
    aggp2 = _sc_seg_sum(h1, src3, dst3)
    out = _tc2(h1, aggp2, invd, W_self2, W_neigh2, b2)
    return out
